# baseline (speedup 1.0000x reference)
import functools

import jax
import jax.numpy as jnp
from jax import lax
from jax.experimental import pallas as pl
from jax.experimental.pallas import tpu as pltpu
from jax.experimental.pallas import tpu_sc as plsc

DIM = 64
N_POS = 5
N_CTX = 25
S = 32
NBUF = 2
NC = 2
NS = 16
NW = NC * NS
LANES = 16


def _sc_dots(v_weight, u_weight, all_idx, B):
    per_w = B // NW
    n_chunks = per_w // S

    mesh = plsc.VectorSubcoreMesh(core_axis_name="c", subcore_axis_name="s")

    @functools.partial(
        pl.kernel,
        mesh=mesh,
        compiler_params=pltpu.CompilerParams(
            use_tc_tiling_on_sc=False, needs_layout_passes=False),
        out_type=jax.ShapeDtypeStruct((B, 2 * LANES), jnp.float32),
        scratch_types=[
            pltpu.VMEM((NBUF, 1 + N_CTX, S), jnp.int32),
            pltpu.VMEM((NBUF, S, DIM), jnp.float32),
            pltpu.VMEM((NBUF, N_CTX, S, DIM), jnp.float32),
            pltpu.VMEM((NBUF, S, 2 * LANES), jnp.float32),
            pltpu.SemaphoreType.DMA,
            pltpu.SemaphoreType.DMA,
            pltpu.SemaphoreType.DMA,
            pltpu.SemaphoreType.DMA,
        ],
    )
    def sc_kernel(v_hbm, u_hbm, idx_hbm, out_hbm,
                  idx_v, v_buf, u_buf, dots, sg0, sg1, so0, so1):
        wid = lax.axis_index("s") * NC + lax.axis_index("c")
        base = wid * per_w
        sgs = (sg0, sg1)
        sos = (so0, so1)

        def issue(chunk, b):
            off = base + chunk * S
            pltpu.sync_copy(idx_hbm.at[:, pl.ds(off, S)], idx_v.at[b])
            pltpu.async_copy(v_hbm.at[idx_v.at[b, 0]], v_buf.at[b], sgs[b])
            for j in range(N_CTX):
                pltpu.async_copy(u_hbm.at[idx_v.at[b, 1 + j]],
                                 u_buf.at[b, j], sgs[b])

        def drain_gathers(b):
            pltpu.make_async_copy(v_hbm.at[idx_v.at[b, 0]],
                                  v_buf.at[b], sgs[b]).wait()
            for j in range(N_CTX):
                pltpu.make_async_copy(u_hbm.at[idx_v.at[b, 1 + j]],
                                      u_buf.at[b, j], sgs[b]).wait()

        def compute(b):
            def body_s(s, carry):
                c = [v_buf[b, s, pl.ds(t * LANES, LANES)] for t in range(4)]
                lane = lax.iota(jnp.int32, LANES)
                dlo = jnp.zeros((LANES,), jnp.float32)
                dhi = jnp.full((LANES,), 30.0, jnp.float32)
                for j in range(N_CTX):
                    acc = u_buf[b, j, s, pl.ds(0, LANES)] * c[0]
                    for t in range(1, 4):
                        acc = acc + u_buf[b, j, s, pl.ds(t * LANES, LANES)] * c[t]
                    d = jnp.sum(acc)
                    d = d if j < N_POS else -d
                    if j < LANES:
                        dlo = jnp.where(lane == j, d, dlo)
                    else:
                        dhi = jnp.where(lane == (j - LANES), d, dhi)
                dots[b, s, pl.ds(0, LANES)] = dlo
                dots[b, s, pl.ds(LANES, LANES)] = dhi
                return carry
            lax.fori_loop(0, S, body_s, 0)

        issue(0, 0)

        def outer(i, carry):
            for b in range(NBUF):
                chunk = NBUF * i + b
                nb = 1 - b

                @pl.when(chunk + 1 < n_chunks)
                def _():
                    issue(chunk + 1, nb)

                drain_gathers(b)

                @pl.when(chunk >= NBUF)
                def _():
                    pltpu.make_async_copy(
                        dots.at[b], out_hbm.at[pl.ds(0, S)], sos[b]).wait()

                compute(b)
                off = base + chunk * S
                pltpu.async_copy(dots.at[b], out_hbm.at[pl.ds(off, S)],
                                 sos[b])
            return carry

        lax.fori_loop(0, n_chunks // NBUF, outer, 0)

        for b in range(NBUF):
            pltpu.make_async_copy(dots.at[b], out_hbm.at[pl.ds(0, S)],
                                  sos[b]).wait()

    return sc_kernel(v_weight, u_weight, all_idx)


def _loss_body(dots_ref, out_ref):
    x = dots_ref[...]
    y = jax.nn.log_sigmoid(x)
    out_ref[...] = -jnp.sum(y, axis=1, keepdims=True)


def kernel(center_words, positive_words, negative_words, v_weight, u_weight):
    B = center_words.shape[0]
    all_idx = jnp.concatenate(
        [center_words[None, :], positive_words.T, negative_words.T], axis=0)
    all_idx = all_idx.astype(jnp.int32)

    dots = _sc_dots(v_weight, u_weight, all_idx, B)

    bt = 4096
    loss2d = pl.pallas_call(
        _loss_body,
        grid=(B // bt,),
        in_specs=[pl.BlockSpec((bt, 2 * LANES), lambda i: (i, 0))],
        out_specs=pl.BlockSpec((bt, 1), lambda i: (i, 0)),
        out_shape=jax.ShapeDtypeStruct((B, 1), jnp.float32),
    )(dots)
    return loss2d[:, 0]
